# baseline (speedup 1.0000x reference)
import jax
import jax.numpy as jnp
from jax.experimental import pallas as pl
from jax.experimental.pallas import tpu as pltpu

_T, _B, _N, _I = 12, 16, 512, 2
_H = 64
_L = 2
_S = 2
_K = 3
_NUM_MAT = 1 + _S * (_K - 1)
_P = 8
_Q = 4
_C0 = _I + _H
_C1 = _H + _H
_PH = 2 * _H
_G = 6 * _H


def _diffuse(s1, s2, xb):
    t1ab = jnp.dot(s1, xb, preferred_element_type=jnp.float32
                   ).astype(jnp.bfloat16)
    t2ab = (2.0 * jnp.dot(s1, t1ab, preferred_element_type=jnp.float32)
            ).astype(jnp.bfloat16) - xb
    t1bb = jnp.dot(s2, xb, preferred_element_type=jnp.float32
                   ).astype(jnp.bfloat16)
    t2bb = (2.0 * jnp.dot(s2, t1bb, preferred_element_type=jnp.float32)
            ).astype(jnp.bfloat16) - xb
    return [xb, t1ab, t2ab, t1bb, t2bb]


def _proj0(mats, w):
    d = [jnp.dot(m, w[k * _PH:(k + 1) * _PH],
                 preferred_element_type=jnp.float32)
         for k, m in enumerate(mats)]
    return (d[0] + d[1]) + (d[2] + d[3]) + d[4]


def _proj(mats, w, acc):
    return acc + _proj0(mats, w)


def _bf(x):
    return x.astype(jnp.bfloat16)


def _body(x_ref, sup_ref, w0i_ref, wgi0_ref, wc0_ref,
          wg1_ref, wc1_ref, b0_ref, b1_ref,
          out_ref, g0c_ref):
    s1m = sup_ref[0]
    s2m = sup_ref[1]
    w0i = w0i_ref[:, :]
    wgi0 = wgi0_ref[:, :]
    wc0 = wc0_ref[:, :]
    wg1 = wg1_ref[:, :]
    wc1 = wc1_ref[:, :]
    b0c = b0_ref[:, :]
    b1c = b1_ref[:, :]

    imats = _diffuse(s1m, s2m, _bf(x_ref[0]))
    for t in range(_T):
        cols = jnp.concatenate(
            [m[:, _P * _I * t:_P * _I * (t + 1)] for m in imats], axis=1)
        g0c_ref[t] = _bf(jnp.dot(cols, w0i,
                                 preferred_element_type=jnp.float32) + b0c)
    g0c_ref[_T] = jnp.zeros((_N, _Q * _G), jnp.bfloat16)

    g00 = g0c_ref[0].astype(jnp.float32)
    o00 = tuple(
        (1.0 - jax.nn.sigmoid(g00[:, q * _G + _PH:q * _G + 2 * _PH]))
        * jnp.tanh(g00[:, q * _G + 2 * _PH:(q + 1) * _G])
        for q in range(_Q))

    def step(t, carry):
        o0s, s1s, _ = carry

        osb = jnp.concatenate([_bf(a) for a in o0s]
                              + [_bf(a) for a in s1s], axis=1)
        dmats = _diffuse(s1m, s2m, osb)
        g0n = g0c_ref[t + 1]
        qph = _Q * _PH

        rc0s, rc1s, u1s, u0s, gis = [], [], [], [], []
        for q in range(_Q):
            dm0 = [m[:, q * _PH:(q + 1) * _PH] for m in dmats]
            dm1 = [m[:, qph + q * _PH:qph + (q + 1) * _PH] for m in dmats]
            fused = _proj0(dm0, wgi0)
            gi = fused[:, :3 * _PH] + b1c
            g1 = jax.nn.sigmoid(_proj(dm1, wg1, gi[:, :2 * _PH]))
            g0 = jax.nn.sigmoid(
                fused[:, 3 * _PH:]
                + g0n[:, q * _G:q * _G + 2 * _PH].astype(jnp.float32))
            rc0s.append(g0[:, :_PH] * o0s[q])
            rc1s.append(g1[:, :_PH] * s1s[q])
            u1s.append(g1[:, _PH:])
            u0s.append(g0[:, _PH:])
            gis.append(gi)

        rcb = jnp.concatenate([_bf(a) for a in rc0s]
                              + [_bf(a) for a in rc1s], axis=1)
        rcmats = _diffuse(s1m, s2m, rcb)

        o0n, s1n = [], []
        for q in range(_Q):
            cm0 = [m[:, q * _PH:(q + 1) * _PH] for m in rcmats]
            cm1 = [m[:, qph + q * _PH:qph + (q + 1) * _PH]
                   for m in rcmats]
            cand0 = jnp.tanh(_proj(
                cm0, wc0,
                g0n[:, q * _G + 2 * _PH:(q + 1) * _G]
                .astype(jnp.float32)))
            cand1 = jnp.tanh(_proj(cm1, wc1, gis[q][:, 2 * _PH:]))
            o0n.append(u0s[q] * o0s[q] + (1.0 - u0s[q]) * cand0)
            s1n.append(u1s[q] * s1s[q] + (1.0 - u1s[q]) * cand1)

        return (tuple(o0n), tuple(s1n), o0s)

    z = tuple(jnp.zeros((_N, _PH), jnp.float32) for _ in range(_Q))
    _, s1_fin, s0_fin = jax.lax.fori_loop(0, _T, step, (o00, z, z))
    out_ref[0, 0] = jnp.concatenate(list(s0_fin), axis=1)
    out_ref[1, 0] = jnp.concatenate(list(s1_fin), axis=1)


def _bd_gate(w):
    r, u = w[:, :_H], w[:, _H:]
    z = jnp.zeros_like(r)
    return jnp.concatenate(
        [jnp.concatenate([r, z, u, z], axis=1),
         jnp.concatenate([z, r, z, u], axis=1)], axis=0)


def _bd_cand(w):
    z = jnp.zeros_like(w)
    return jnp.concatenate(
        [jnp.concatenate([w, z], axis=1),
         jnp.concatenate([z, w], axis=1)], axis=0)


def _bd_fused(wr, wh):
    r, u = wr[:, :_H], wr[:, _H:]
    z = jnp.zeros_like(r)
    return jnp.concatenate(
        [jnp.concatenate([r, z, u, z, wh, z], axis=1),
         jnp.concatenate([z, r, z, u, z, wh], axis=1)], axis=0)


def kernel(inputs, supports, W_ru_0, b_ru_0, W_h_0, b_h_0,
           W_ru_1, b_ru_1, W_h_1, b_h_1):
    x_g = inputs.transpose(1, 2, 0, 3)
    x_g = x_g.reshape(_B // _P, _P, _N, _T, _I)
    x_g = x_g.transpose(0, 2, 3, 1, 4).reshape(_B // _P, _N, _T * _P * _I)

    wg0 = jnp.concatenate(
        [_bd_gate(W_ru_0[k * _C0 + _I:(k + 1) * _C0])
         for k in range(_NUM_MAT)], axis=0)
    wc0 = jnp.concatenate(
        [_bd_cand(W_h_0[k * _C0 + _I:(k + 1) * _C0])
         for k in range(_NUM_MAT)], axis=0)
    wg1 = jnp.concatenate(
        [_bd_gate(W_ru_1[k * _C1 + _H:(k + 1) * _C1])
         for k in range(_NUM_MAT)], axis=0)
    wc1 = jnp.concatenate(
        [_bd_cand(W_h_1[k * _C1 + _H:(k + 1) * _C1])
         for k in range(_NUM_MAT)], axis=0)
    w1i = jnp.concatenate(
        [_bd_fused(W_ru_1[k * _C1:k * _C1 + _H],
                   W_h_1[k * _C1:k * _C1 + _H])
         for k in range(_NUM_MAT)], axis=0)
    wgi0 = jnp.concatenate(
        [jnp.concatenate([w1i[k * _PH:(k + 1) * _PH],
                          wg0[k * _PH:(k + 1) * _PH]], axis=1)
         for k in range(_NUM_MAT)], axis=0)

    blocks = []
    for k in range(_NUM_MAT):
        r = W_ru_0[k * _C0:k * _C0 + _I, :_H]
        u = W_ru_0[k * _C0:k * _C0 + _I, _H:]
        c = W_h_0[k * _C0:k * _C0 + _I]
        z = jnp.zeros_like(r)
        pair = jnp.concatenate(
            [jnp.concatenate([r, z, u, z, c, z], axis=1),
             jnp.concatenate([z, r, z, u, z, c], axis=1)], axis=0)
        zz = jnp.zeros_like(pair)
        blocks.append(jnp.concatenate(
            [jnp.concatenate([pair if i == q else zz for i in range(_Q)],
                             axis=1) for q in range(_Q)], axis=0))
    w0i = jnp.concatenate(blocks, axis=0)

    b0q = jnp.concatenate([b_ru_0[:_H], b_ru_0[:_H], b_ru_0[_H:],
                           b_ru_0[_H:], b_h_0, b_h_0])
    b0c = jnp.concatenate([b0q] * _Q).reshape(1, _Q * 6 * _H)
    b1c = jnp.concatenate([b_ru_1[:_H], b_ru_1[:_H], b_ru_1[_H:],
                           b_ru_1[_H:], b_h_1, b_h_1]).reshape(1, 6 * _H)

    out = pl.pallas_call(
        _body,
        grid=(_B // _P,),
        in_specs=[
            pl.BlockSpec((1, _N, _T * _P * _I), lambda p: (p, 0, 0)),
            pl.BlockSpec((_S, _N, _N), lambda p: (0, 0, 0)),
            pl.BlockSpec((_NUM_MAT * _P * _I, _Q * 6 * _H),
                         lambda p: (0, 0)),
            pl.BlockSpec((_NUM_MAT * 2 * _H, 10 * _H), lambda p: (0, 0)),
            pl.BlockSpec((_NUM_MAT * 2 * _H, 2 * _H), lambda p: (0, 0)),
            pl.BlockSpec((_NUM_MAT * 2 * _H, 4 * _H), lambda p: (0, 0)),
            pl.BlockSpec((_NUM_MAT * 2 * _H, 2 * _H), lambda p: (0, 0)),
            pl.BlockSpec((1, _Q * 6 * _H), lambda p: (0, 0)),
            pl.BlockSpec((1, 6 * _H), lambda p: (0, 0)),
        ],
        out_specs=pl.BlockSpec((_L, 1, _N, _P * _H), lambda p: (0, p, 0, 0)),
        out_shape=jax.ShapeDtypeStruct((_L, _B // _P, _N, _P * _H),
                                       jnp.float32),
        scratch_shapes=[pltpu.VMEM((_T + 1, _N, _Q * 6 * _H),
                                   jnp.bfloat16)],
        compiler_params=pltpu.CompilerParams(
            dimension_semantics=("parallel",)),
    )(x_g, supports.astype(jnp.bfloat16),
      w0i.astype(jnp.bfloat16), wgi0.astype(jnp.bfloat16),
      wc0.astype(jnp.bfloat16),
      wg1.astype(jnp.bfloat16), wc1.astype(jnp.bfloat16), b0c, b1c)
    out = out.reshape(_L, _B // _P, _N, _P, _H)
    out = out.transpose(0, 1, 3, 2, 4).reshape(_L, _B, _N, _H)
    return out


# speedup vs baseline: 1.0631x; 1.0631x over previous
import jax
import jax.numpy as jnp
from jax.experimental import pallas as pl
from jax.experimental.pallas import tpu as pltpu

_T, _B, _N, _I = 12, 16, 512, 2
_H = 64
_L = 2
_S = 2
_K = 3
_NUM_MAT = 1 + _S * (_K - 1)
_P = 8
_Q = 4
_C0 = _I + _H
_C1 = _H + _H
_PH = 2 * _H
_G = 6 * _H


def _diffuse(s1, s2, xb):
    t1ab = jnp.dot(s1, xb, preferred_element_type=jnp.float32
                   ).astype(jnp.bfloat16)
    t2ab = (2.0 * jnp.dot(s1, t1ab, preferred_element_type=jnp.float32)
            ).astype(jnp.bfloat16) - xb
    t1bb = jnp.dot(s2, xb, preferred_element_type=jnp.float32
                   ).astype(jnp.bfloat16)
    t2bb = (2.0 * jnp.dot(s2, t1bb, preferred_element_type=jnp.float32)
            ).astype(jnp.bfloat16) - xb
    return [xb, t1ab, t2ab, t1bb, t2bb]


def _proj(mats, w, acc):
    d = [jnp.dot(m, w[k * _PH:(k + 1) * _PH],
                 preferred_element_type=jnp.float32)
         for k, m in enumerate(mats)]
    return ((acc + d[0]) + (d[1] + d[2])) + (d[3] + d[4])


def _bf(x):
    return x.astype(jnp.bfloat16)


def _body(x_ref, sup_ref, w0i_ref, wg0_ref, wc0_ref,
          w1i_ref, wg1_ref, wc1_ref, b0_ref, b1_ref,
          out_ref, g0c_ref):
    s1m = sup_ref[0]
    s2m = sup_ref[1]
    w0i = w0i_ref[:, :]
    wg0 = wg0_ref[:, :]
    wc0 = wc0_ref[:, :]
    w1i = w1i_ref[:, :]
    wg1 = wg1_ref[:, :]
    wc1 = wc1_ref[:, :]
    b0c = b0_ref[:, :]
    b1c = b1_ref[:, :]

    imats = _diffuse(s1m, s2m, _bf(x_ref[0]))
    for t in range(_T):
        cols = jnp.concatenate(
            [m[:, _P * _I * t:_P * _I * (t + 1)] for m in imats], axis=1)
        g0c_ref[t] = _bf(jnp.dot(cols, w0i,
                                 preferred_element_type=jnp.float32) + b0c)
    g0c_ref[_T] = jnp.zeros((_N, _Q * _G), jnp.bfloat16)

    g00 = g0c_ref[0].astype(jnp.float32)
    o00 = tuple(
        (1.0 - jax.nn.sigmoid(g00[:, q * _G + _PH:q * _G + 2 * _PH]))
        * jnp.tanh(g00[:, q * _G + 2 * _PH:(q + 1) * _G])
        for q in range(_Q))

    def step(t, carry):
        o0s, s1s, _ = carry

        osb = jnp.concatenate([_bf(a) for a in o0s]
                              + [_bf(a) for a in s1s], axis=1)
        dmats = _diffuse(s1m, s2m, osb)
        g0n = g0c_ref[t + 1]
        qph = _Q * _PH

        rc0s, rc1s, u1s, u0s, gis = [], [], [], [], []
        for q in range(_Q):
            dm0 = [m[:, q * _PH:(q + 1) * _PH] for m in dmats]
            dm1 = [m[:, qph + q * _PH:qph + (q + 1) * _PH] for m in dmats]
            gi = _proj(dm0, w1i, b1c)
            g1 = jax.nn.sigmoid(_proj(dm1, wg1, gi[:, :2 * _PH]))
            g0 = jax.nn.sigmoid(_proj(
                dm0, wg0,
                g0n[:, q * _G:q * _G + 2 * _PH].astype(jnp.float32)))
            rc0s.append(g0[:, :_PH] * o0s[q])
            rc1s.append(g1[:, :_PH] * s1s[q])
            u1s.append(g1[:, _PH:])
            u0s.append(g0[:, _PH:])
            gis.append(gi)

        rcb = jnp.concatenate([_bf(a) for a in rc0s]
                              + [_bf(a) for a in rc1s], axis=1)
        rcmats = _diffuse(s1m, s2m, rcb)

        o0n, s1n = [], []
        for q in range(_Q):
            cm0 = [m[:, q * _PH:(q + 1) * _PH] for m in rcmats]
            cm1 = [m[:, qph + q * _PH:qph + (q + 1) * _PH]
                   for m in rcmats]
            cand0 = jnp.tanh(_proj(
                cm0, wc0,
                g0n[:, q * _G + 2 * _PH:(q + 1) * _G]
                .astype(jnp.float32)))
            cand1 = jnp.tanh(_proj(cm1, wc1, gis[q][:, 2 * _PH:]))
            o0n.append(u0s[q] * o0s[q] + (1.0 - u0s[q]) * cand0)
            s1n.append(u1s[q] * s1s[q] + (1.0 - u1s[q]) * cand1)

        return (tuple(o0n), tuple(s1n), o0s)

    z = tuple(jnp.zeros((_N, _PH), jnp.float32) for _ in range(_Q))
    _, s1_fin, s0_fin = jax.lax.fori_loop(0, _T, step, (o00, z, z))
    out_ref[0, 0] = jnp.concatenate(list(s0_fin), axis=1)
    out_ref[1, 0] = jnp.concatenate(list(s1_fin), axis=1)


def _bd_gate(w):
    r, u = w[:, :_H], w[:, _H:]
    z = jnp.zeros_like(r)
    return jnp.concatenate(
        [jnp.concatenate([r, z, u, z], axis=1),
         jnp.concatenate([z, r, z, u], axis=1)], axis=0)


def _bd_cand(w):
    z = jnp.zeros_like(w)
    return jnp.concatenate(
        [jnp.concatenate([w, z], axis=1),
         jnp.concatenate([z, w], axis=1)], axis=0)


def _bd_fused(wr, wh):
    r, u = wr[:, :_H], wr[:, _H:]
    z = jnp.zeros_like(r)
    return jnp.concatenate(
        [jnp.concatenate([r, z, u, z, wh, z], axis=1),
         jnp.concatenate([z, r, z, u, z, wh], axis=1)], axis=0)


def kernel(inputs, supports, W_ru_0, b_ru_0, W_h_0, b_h_0,
           W_ru_1, b_ru_1, W_h_1, b_h_1):
    x_g = inputs.transpose(1, 2, 0, 3)
    x_g = x_g.reshape(_B // _P, _P, _N, _T, _I)
    x_g = x_g.transpose(0, 2, 3, 1, 4).reshape(_B // _P, _N, _T * _P * _I)

    wg0 = jnp.concatenate(
        [_bd_gate(W_ru_0[k * _C0 + _I:(k + 1) * _C0])
         for k in range(_NUM_MAT)], axis=0)
    wc0 = jnp.concatenate(
        [_bd_cand(W_h_0[k * _C0 + _I:(k + 1) * _C0])
         for k in range(_NUM_MAT)], axis=0)
    wg1 = jnp.concatenate(
        [_bd_gate(W_ru_1[k * _C1 + _H:(k + 1) * _C1])
         for k in range(_NUM_MAT)], axis=0)
    wc1 = jnp.concatenate(
        [_bd_cand(W_h_1[k * _C1 + _H:(k + 1) * _C1])
         for k in range(_NUM_MAT)], axis=0)
    w1i = jnp.concatenate(
        [_bd_fused(W_ru_1[k * _C1:k * _C1 + _H],
                   W_h_1[k * _C1:k * _C1 + _H])
         for k in range(_NUM_MAT)], axis=0)

    blocks = []
    for k in range(_NUM_MAT):
        r = W_ru_0[k * _C0:k * _C0 + _I, :_H]
        u = W_ru_0[k * _C0:k * _C0 + _I, _H:]
        c = W_h_0[k * _C0:k * _C0 + _I]
        z = jnp.zeros_like(r)
        pair = jnp.concatenate(
            [jnp.concatenate([r, z, u, z, c, z], axis=1),
             jnp.concatenate([z, r, z, u, z, c], axis=1)], axis=0)
        zz = jnp.zeros_like(pair)
        blocks.append(jnp.concatenate(
            [jnp.concatenate([pair if i == q else zz for i in range(_Q)],
                             axis=1) for q in range(_Q)], axis=0))
    w0i = jnp.concatenate(blocks, axis=0)

    b0q = jnp.concatenate([b_ru_0[:_H], b_ru_0[:_H], b_ru_0[_H:],
                           b_ru_0[_H:], b_h_0, b_h_0])
    b0c = jnp.concatenate([b0q] * _Q).reshape(1, _Q * 6 * _H)
    b1c = jnp.concatenate([b_ru_1[:_H], b_ru_1[:_H], b_ru_1[_H:],
                           b_ru_1[_H:], b_h_1, b_h_1]).reshape(1, 6 * _H)

    out = pl.pallas_call(
        _body,
        grid=(_B // _P,),
        in_specs=[
            pl.BlockSpec((1, _N, _T * _P * _I), lambda p: (p, 0, 0)),
            pl.BlockSpec((_S, _N, _N), lambda p: (0, 0, 0)),
            pl.BlockSpec((_NUM_MAT * _P * _I, _Q * 6 * _H),
                         lambda p: (0, 0)),
            pl.BlockSpec((_NUM_MAT * 2 * _H, 4 * _H), lambda p: (0, 0)),
            pl.BlockSpec((_NUM_MAT * 2 * _H, 2 * _H), lambda p: (0, 0)),
            pl.BlockSpec((_NUM_MAT * 2 * _H, 6 * _H), lambda p: (0, 0)),
            pl.BlockSpec((_NUM_MAT * 2 * _H, 4 * _H), lambda p: (0, 0)),
            pl.BlockSpec((_NUM_MAT * 2 * _H, 2 * _H), lambda p: (0, 0)),
            pl.BlockSpec((1, _Q * 6 * _H), lambda p: (0, 0)),
            pl.BlockSpec((1, 6 * _H), lambda p: (0, 0)),
        ],
        out_specs=pl.BlockSpec((_L, 1, _N, _P * _H), lambda p: (0, p, 0, 0)),
        out_shape=jax.ShapeDtypeStruct((_L, _B // _P, _N, _P * _H),
                                       jnp.float32),
        scratch_shapes=[pltpu.VMEM((_T + 1, _N, _Q * 6 * _H),
                                   jnp.bfloat16)],
        compiler_params=pltpu.CompilerParams(
            dimension_semantics=("parallel",)),
    )(x_g, supports.astype(jnp.bfloat16),
      w0i.astype(jnp.bfloat16), wg0.astype(jnp.bfloat16),
      wc0.astype(jnp.bfloat16), w1i.astype(jnp.bfloat16),
      wg1.astype(jnp.bfloat16), wc1.astype(jnp.bfloat16), b0c, b1c)
    out = out.reshape(_L, _B // _P, _N, _P, _H)
    out = out.transpose(0, 1, 3, 2, 4).reshape(_L, _B, _N, _H)
    return out
